# baseline (speedup 1.0000x reference)
import functools

import jax
import jax.numpy as jnp
from jax import lax
from jax.experimental import pallas as pl
from jax.experimental.pallas import tpu as pltpu
from jax.experimental.pallas import tpu_sc as plsc

B = 16384
L = 200
VOCAB = 1000
D = 64
KP = 1024

NC = 2
NS = 16
NW = NC * NS
ROWS_PER_W = B // NW
CHUNK = 32
NCHUNKS = ROWS_PER_W // CHUNK
NPAIR = NCHUNKS // 2

_mesh = plsc.VectorSubcoreMesh(core_axis_name="c", subcore_axis_name="s")


@functools.partial(
    pl.kernel,
    mesh=_mesh,
    out_type=jax.ShapeDtypeStruct((B, KP), jnp.float32),
    scratch_types=[
        pltpu.VMEM((2, CHUNK, L), jnp.int32),
        pltpu.VMEM((2, CHUNK, KP), jnp.float32),
        pltpu.SemaphoreType.DMA,
        pltpu.SemaphoreType.DMA,
    ],
    compiler_params=pltpu.CompilerParams(needs_layout_passes=False),
)
def _hist(idx_hbm, cnt_hbm, idx_v, cnt_v, sem0, sem1):
    wid = lax.axis_index("s") * NC + lax.axis_index("c")
    row_base_w = wid * ROWS_PER_W
    zero16 = jnp.zeros((16,), jnp.float32)
    ones = jnp.full((16,), 1.0, jnp.float32)
    elig = lax.iota(jnp.int32, 16) >= 8
    sems = (sem0, sem1)

    _OFFS = tuple(s * 16 for s in range(L // 16)) + (L - 16,)

    def zero_old(b):
        bvec = jnp.full((16,), b, jnp.int32)

        def zrow(r, c):
            rvec = jnp.full((16,), 0, jnp.int32) + r
            for off in _OFFS:
                ivec = idx_v[b, r, pl.ds(off, 16)]
                plsc.store_scatter(cnt_v, [bvec, rvec, ivec], zero16)
            return c

        lax.fori_loop(0, CHUNK, zrow, 0)

    def fill_chunk(ci, b):
        base = row_base_w + ci * CHUNK
        pltpu.sync_copy(idx_hbm.at[pl.ds(base, CHUNK)], idx_v.at[b])
        bvec = jnp.full((16,), b, jnp.int32)

        def row_body(r, carry2):
            rvec = jnp.full((16,), 0, jnp.int32) + r
            for off in _OFFS[:-1]:
                ivec = idx_v[b, r, pl.ds(off, 16)]
                plsc.addupdate_scatter(cnt_v, [bvec, rvec, ivec], ones)
            ivec = idx_v[b, r, pl.ds(_OFFS[-1], 16)]
            plsc.addupdate_scatter(cnt_v, [bvec, rvec, ivec], ones,
                                   mask=elig)
            return carry2

        lax.fori_loop(0, CHUNK, row_body, 0)
        return base

    def start_out(ci, b):
        base = row_base_w + ci * CHUNK
        pltpu.async_copy(cnt_v.at[b], cnt_hbm.at[pl.ds(base, CHUNK)],
                         sems[b])

    def wait_out(ci, b):
        base = row_base_w + ci * CHUNK
        pltpu.make_async_copy(cnt_v.at[b],
                              cnt_hbm.at[pl.ds(base, CHUNK)],
                              sems[b]).wait()

    def clear_body(i, c):
        for bb in range(2):
            for v in range(16):
                cnt_v[bb, i // (KP // 256),
                      pl.ds((i % (KP // 256) * 16 + v) * 16, 16)] = zero16
        return c

    lax.fori_loop(0, CHUNK * KP // 256, clear_body, 0)

    for b in range(2):
        fill_chunk(b, b)
        start_out(b, b)

    def pair_body(g, carry):
        for b in range(2):
            ci = g * 2 + b
            wait_out(ci - 2, b)
            zero_old(b)
            fill_chunk(ci, b)
            start_out(ci, b)
        return carry

    lax.fori_loop(1, NPAIR, pair_body, 0)
    for b in range(2):
        wait_out(NCHUNKS - 2 + b, b)


TM = 2048


def _mm_body(c_ref, t_ref, o_ref):
    o_ref[...] = jnp.dot(
        c_ref[...].astype(jnp.bfloat16),
        t_ref[...],
        preferred_element_type=jnp.float32,
    )


_mm = pl.pallas_call(
    _mm_body,
    grid=(B // TM,),
    in_specs=[
        pl.BlockSpec((TM, KP), lambda i: (i, 0)),
        pl.BlockSpec((KP, D), lambda i: (0, 0)),
    ],
    out_specs=pl.BlockSpec((TM, D), lambda i: (i, 0)),
    out_shape=jax.ShapeDtypeStruct((B, D), jnp.float32),
)


def kernel(indices, table):
    counts = _hist(indices)
    tab_pad = jnp.zeros((KP, D), jnp.bfloat16).at[:VOCAB].set(
        table.astype(jnp.bfloat16))
    return _mm(counts, tab_pad)


# speedup vs baseline: 1.1794x; 1.1794x over previous
import functools

import jax
import jax.numpy as jnp
from jax import lax
from jax.experimental import pallas as pl
from jax.experimental.pallas import tpu as pltpu
from jax.experimental.pallas import tpu_sc as plsc

B = 16384
L = 200
VOCAB = 1000
D = 64
KP = 1024

NC = 2
NS = 16
NW = NC * NS
ROWS_PER_W = B // NW
CHUNK = 32
NCHUNKS = ROWS_PER_W // CHUNK
NPAIR = NCHUNKS // 2

_mesh = plsc.VectorSubcoreMesh(core_axis_name="c", subcore_axis_name="s")


@functools.partial(
    pl.kernel,
    mesh=_mesh,
    out_type=jax.ShapeDtypeStruct((B, KP), jnp.float32),
    scratch_types=[
        pltpu.VMEM((2, CHUNK, L), jnp.int32),
        pltpu.VMEM((2, CHUNK, KP), jnp.float32),
        pltpu.SemaphoreType.DMA,
        pltpu.SemaphoreType.DMA,
    ],
    compiler_params=pltpu.CompilerParams(needs_layout_passes=False),
)
def _hist(idx_hbm, cnt_hbm, idx_v, cnt_v, sem0, sem1):
    wid = lax.axis_index("s") * NC + lax.axis_index("c")
    row_base_w = wid * ROWS_PER_W
    zero16 = jnp.zeros((16,), jnp.float32)
    ones = jnp.full((16,), 1.0, jnp.float32)
    elig = lax.iota(jnp.int32, 16) >= 8
    sems = (sem0, sem1)

    _OFFS = tuple(s * 16 for s in range(L // 16)) + (L - 16,)

    def zero_old(b):
        def clear_row(i, c):
            for v in range(KP // 16):
                cnt_v[b, i, pl.ds(v * 16, 16)] = zero16
            return c

        lax.fori_loop(0, CHUNK, clear_row, 0)

    def fill_chunk(ci, b):
        base = row_base_w + ci * CHUNK
        pltpu.sync_copy(idx_hbm.at[pl.ds(base, CHUNK)], idx_v.at[b])
        bvec = jnp.full((16,), b, jnp.int32)

        def row_body(r, carry2):
            rvec = jnp.full((16,), 0, jnp.int32) + r
            for off in _OFFS[:-1]:
                ivec = idx_v[b, r, pl.ds(off, 16)]
                plsc.addupdate_scatter(cnt_v, [bvec, rvec, ivec], ones)
            ivec = idx_v[b, r, pl.ds(_OFFS[-1], 16)]
            plsc.addupdate_scatter(cnt_v, [bvec, rvec, ivec], ones,
                                   mask=elig)
            return carry2

        lax.fori_loop(0, CHUNK, row_body, 0)
        return base

    def start_out(ci, b):
        base = row_base_w + ci * CHUNK
        pltpu.async_copy(cnt_v.at[b], cnt_hbm.at[pl.ds(base, CHUNK)],
                         sems[b])

    def wait_out(ci, b):
        base = row_base_w + ci * CHUNK
        pltpu.make_async_copy(cnt_v.at[b],
                              cnt_hbm.at[pl.ds(base, CHUNK)],
                              sems[b]).wait()

    def clear_body(i, c):
        for bb in range(2):
            for v in range(16):
                cnt_v[bb, i // (KP // 256),
                      pl.ds((i % (KP // 256) * 16 + v) * 16, 16)] = zero16
        return c

    lax.fori_loop(0, CHUNK * KP // 256, clear_body, 0)

    for b in range(2):
        fill_chunk(b, b)
        start_out(b, b)

    def pair_body(g, carry):
        for b in range(2):
            ci = g * 2 + b
            wait_out(ci - 2, b)
            zero_old(b)
            fill_chunk(ci, b)
            start_out(ci, b)
        return carry

    lax.fori_loop(1, NPAIR, pair_body, 0)
    for b in range(2):
        wait_out(NCHUNKS - 2 + b, b)


TM = 2048


def _mm_body(c_ref, t_ref, o_ref):
    o_ref[...] = jnp.dot(
        c_ref[...].astype(jnp.bfloat16),
        t_ref[...],
        preferred_element_type=jnp.float32,
    )


_mm = pl.pallas_call(
    _mm_body,
    grid=(B // TM,),
    in_specs=[
        pl.BlockSpec((TM, KP), lambda i: (i, 0)),
        pl.BlockSpec((KP, D), lambda i: (0, 0)),
    ],
    out_specs=pl.BlockSpec((TM, D), lambda i: (i, 0)),
    out_shape=jax.ShapeDtypeStruct((B, D), jnp.float32),
)


def kernel(indices, table):
    counts = _hist(indices)
    tab_pad = jnp.zeros((KP, D), jnp.bfloat16).at[:VOCAB].set(
        table.astype(jnp.bfloat16))
    return _mm(counts, tab_pad)
